# Initial kernel scaffold; baseline (speedup 1.0000x reference)
import jax
import jax.numpy as jnp
from jax.experimental import pallas as pl


def kernel(h_user, h_item, edge_index_user_buys_item, edge_index_item_bought_by_user, W_ub, b_ub, W_bu, b_bu, g_user, be_user, g_item, be_item):
    raise NotImplementedError("write your pallas kernel here")


# speedup vs baseline: 1.1960x; 1.1960x over previous
import functools

import jax
import jax.numpy as jnp
from jax import lax
from jax.experimental import pallas as pl
from jax.experimental.pallas import tpu as pltpu
from jax.experimental.pallas import tpu_sc as plsc

N = 50000
D = 128
E = 500000
NS = 16
CH = 128
CPW = 248
EPW = CPW * CH
EPAD = NS * EPW
NP = D // 16
ACC_R = 50048
ZR = ACC_R // NS
ZB = 391
SB = 782


def _sc_body(hu8, hi8, s_ub, d_ub, s_bu, d_bu,
             sums_i, cnt_i, sums_u, cnt_u,
             acc, cacc, zbuf, ones, sidx, didx, rows, stage, sem):
    core = lax.axis_index("c")
    sub = lax.axis_index("s")

    def fill(ref, n, val):
        def body(i, c):
            ref[i] = jnp.full((16,), val, jnp.float32)
            return c
        lax.fori_loop(0, n, body, 0)

    fill(zbuf, ZB, 0.0)
    fill(ones, CH, 1.0)

    def zero(dst):
        for k in range(ZR // ZB):
            pltpu.sync_copy(zbuf, dst.at[pl.ds(sub * ZR + k * ZB, ZB)])

    def side(tab8, s_e, d_e, sums_o, cnt_o):
        zero(cacc)
        for p in range(NP):
            zero(acc)
            plsc.subcore_barrier()

            tab = tab8.at[p]

            def chunk(j, c):
                be = sub * EPW + j * CH
                pltpu.sync_copy(s_e.at[pl.ds(be, CH)], sidx)
                pltpu.sync_copy(d_e.at[pl.ds(be, CH)], didx)
                pltpu.async_copy(tab.at[sidx], rows, sem).wait()
                pltpu.sync_copy(rows, acc.at[didx], add=True)
                if p == 0:
                    pltpu.sync_copy(ones, cacc.at[didx], add=True)
                return c

            lax.fori_loop(0, CPW, chunk, 0)
            plsc.subcore_barrier()

            def writeout(r0, nchunks, csz):
                for k in range(nchunks):
                    o = r0 + k * csz
                    pltpu.sync_copy(acc.at[pl.ds(o, csz)], stage.at[pl.ds(0, csz)])
                    pltpu.sync_copy(stage.at[pl.ds(0, csz)], sums_o.at[p, pl.ds(o, csz)])
                    if p == 0:
                        pltpu.sync_copy(cacc.at[pl.ds(o, csz)], stage.at[pl.ds(0, csz)])
                        pltpu.sync_copy(stage.at[pl.ds(0, csz)], cnt_o.at[pl.ds(o, csz)])

            pl.when(sub < NS - 1)(lambda: writeout(sub * ZR, 4, SB))
            pl.when(sub == NS - 1)(lambda: writeout((NS - 1) * ZR, 5, 616))
            plsc.subcore_barrier()

    pl.when(core == 0)(lambda: side(hu8, s_ub, d_ub, sums_i, cnt_i))
    pl.when(core == 1)(lambda: side(hi8, s_bu, d_bu, sums_u, cnt_u))


@functools.partial(jax.jit, static_argnames=())
def _sc_aggregate(hu8, hi8, s_ub, d_ub, s_bu, d_bu):
    mesh = plsc.VectorSubcoreMesh(core_axis_name="c", subcore_axis_name="s")
    f32 = jnp.float32
    out_type = [
        jax.ShapeDtypeStruct((NP, N, 16), f32),
        jax.ShapeDtypeStruct((N, 16), f32),
        jax.ShapeDtypeStruct((NP, N, 16), f32),
        jax.ShapeDtypeStruct((N, 16), f32),
    ]
    scratch = [
        pltpu.VMEM_SHARED((ACC_R, 16), f32),
        pltpu.VMEM_SHARED((ACC_R, 16), f32),
        pltpu.VMEM((ZB, 16), f32),
        pltpu.VMEM((CH, 16), f32),
        pltpu.VMEM((CH,), jnp.int32),
        pltpu.VMEM((CH,), jnp.int32),
        pltpu.VMEM((CH, 16), f32),
        pltpu.VMEM((SB, 16), f32),
        pltpu.SemaphoreType.DMA,
    ]
    return pl.kernel(
        _sc_body, out_type=out_type, mesh=mesh, scratch_types=scratch,
        compiler_params=pltpu.CompilerParams(use_tc_tiling_on_sc=False),
        name="het_block_sc_aggregate",
    )(hu8, hi8, s_ub, d_ub, s_bu, d_bu)


def _tc_body(h_ref, sums_ref, cnt_ref, w_ref, b_ref, g_ref, be_ref, out_ref):
    rec = 1.0 / jnp.maximum(cnt_ref[:, 0:1], 1.0)
    y = jnp.dot(h_ref[...], w_ref[0:D, :], preferred_element_type=jnp.float32)
    for p in range(NP):
        y += jnp.dot(sums_ref[p] * rec, w_ref[D + 16 * p:D + 16 * (p + 1), :],
                     preferred_element_type=jnp.float32)
    y = y + b_ref[...]
    mu = jnp.mean(y, axis=-1, keepdims=True)
    yc = y - mu
    var = jnp.mean(yc * yc, axis=-1, keepdims=True)
    out_ref[...] = yc * lax.rsqrt(var + 1e-5) * g_ref[...] + be_ref[...]


def _tc_tail(h, sums, cnt, w, b, g, be):
    R = 2000
    grid = (N // R,)
    return pl.pallas_call(
        _tc_body,
        grid=grid,
        in_specs=[
            pl.BlockSpec((R, D), lambda i: (i, 0)),
            pl.BlockSpec((NP, R, 16), lambda i: (0, i, 0)),
            pl.BlockSpec((R, 16), lambda i: (i, 0)),
            pl.BlockSpec((2 * D, D), lambda i: (0, 0)),
            pl.BlockSpec((1, D), lambda i: (0, 0)),
            pl.BlockSpec((1, D), lambda i: (0, 0)),
            pl.BlockSpec((1, D), lambda i: (0, 0)),
        ],
        out_specs=pl.BlockSpec((R, D), lambda i: (i, 0)),
        out_shape=jax.ShapeDtypeStruct((N, D), jnp.float32),
    )(h, sums, cnt, w, b, g, be)


def kernel(h_user, h_item, edge_index_user_buys_item, edge_index_item_bought_by_user,
           W_ub, b_ub, W_bu, b_bu, g_user, be_user, g_item, be_item):
    i32 = jnp.int32
    pad = EPAD - E
    pad_s = (jnp.arange(pad, dtype=i32) % 64)
    pad_d = N + (jnp.arange(pad, dtype=i32) % 48)

    s_ub = jnp.concatenate([edge_index_user_buys_item[0], pad_s])
    d_ub = jnp.concatenate([edge_index_user_buys_item[1], pad_d])
    s_bu = jnp.concatenate([edge_index_item_bought_by_user[0], pad_s])
    d_bu = jnp.concatenate([edge_index_item_bought_by_user[1], pad_d])

    hu8 = h_user.reshape(N, NP, 16).transpose(1, 0, 2)
    hi8 = h_item.reshape(N, NP, 16).transpose(1, 0, 2)

    sums_i, cnt_i, sums_u, cnt_u = _sc_aggregate(hu8, hi8, s_ub, d_ub, s_bu, d_bu)

    out_item = _tc_tail(h_item, sums_i, cnt_i, W_ub, b_ub.reshape(1, D),
                        g_item.reshape(1, D), be_item.reshape(1, D))
    out_user = _tc_tail(h_user, sums_u, cnt_u, W_bu, b_bu.reshape(1, D),
                        g_user.reshape(1, D), be_user.reshape(1, D))
    return (out_user, out_item)


# speedup vs baseline: 2.0048x; 1.6762x over previous
import functools

import jax
import jax.numpy as jnp
from jax import lax
from jax.experimental import pallas as pl
from jax.experimental.pallas import tpu as pltpu
from jax.experimental.pallas import tpu_sc as plsc

N = 50000
D = 128
E = 500000
NS = 16
CH = 128
CPW = 248
EPW = CPW * CH
EPAD = NS * EPW
NP = D // 16
ACC_R = 50048
ZR = ACC_R // NS
ZB = 391
SB = 782


def _sc_body(hu8, hi8, sd_ub, sd_bu,
             sums_i, cnt_i, sums_u, cnt_u,
             acc, cacc, zbuf, ones, sd0, sd1, rows0, rows1, stage,
             semi0, semi1, semg0, semg1):
    core = lax.axis_index("c")
    sub = lax.axis_index("s")
    sd = (sd0, sd1)
    rows = (rows0, rows1)
    semi = (semi0, semi1)
    semg = (semg0, semg1)

    def fill(ref, n, val):
        def body(i, c):
            ref[i] = jnp.full((16,), val, jnp.float32)
            return c
        lax.fori_loop(0, n, body, 0)

    fill(zbuf, ZB, 0.0)
    fill(ones, CH, 1.0)

    def zero(dst):
        for k in range(ZR // ZB):
            pltpu.sync_copy(zbuf, dst.at[pl.ds(sub * ZR + k * ZB, ZB)])

    def side(tab8, sd_e, sums_o, cnt_o):
        zero(cacc)
        row0 = sub * CPW
        for p in range(NP):
            zero(acc)
            plsc.subcore_barrier()

            tab = tab8.at[p]

            def wait_i(b):
                pltpu.make_async_copy(sd_e.at[row0], sd[b], semi[b]).wait()

            def wait_g(b):
                pltpu.make_async_copy(tab.at[sd[b].at[0]], rows[b], semg[b]).wait()

            def scatter(b):
                pltpu.sync_copy(rows[b], acc.at[sd[b].at[1]], add=True)
                if p == 0:
                    pltpu.sync_copy(ones, cacc.at[sd[b].at[1]], add=True)

            def step(c, b, prefetch, issue_gather):
                wait_g(b)
                scatter(b)
                if prefetch:
                    pltpu.async_copy(sd_e.at[row0 + c + 2], sd[b], semi[b])
                if issue_gather:
                    wait_i(1 - b)
                    pltpu.async_copy(tab.at[sd[1 - b].at[0]], rows[1 - b],
                                     semg[1 - b])

            pltpu.async_copy(sd_e.at[row0], sd0, semi0)
            pltpu.async_copy(sd_e.at[row0 + 1], sd1, semi1)
            wait_i(0)
            pltpu.async_copy(tab.at[sd0.at[0]], rows0, semg0)

            def body(j, c):
                step(2 * j, 0, True, True)
                step(2 * j + 1, 1, True, True)
                return c

            lax.fori_loop(0, (CPW - 2) // 2, body, 0)
            step(CPW - 2, 0, False, True)
            step(CPW - 1, 1, False, False)
            plsc.subcore_barrier()

            def writeout(r0, nchunks, csz):
                for k in range(nchunks):
                    o = r0 + k * csz
                    pltpu.sync_copy(acc.at[pl.ds(o, csz)], stage.at[pl.ds(0, csz)])
                    pltpu.sync_copy(stage.at[pl.ds(0, csz)], sums_o.at[p, pl.ds(o, csz)])
                    if p == 0:
                        pltpu.sync_copy(cacc.at[pl.ds(o, csz)], stage.at[pl.ds(0, csz)])
                        pltpu.sync_copy(stage.at[pl.ds(0, csz)], cnt_o.at[pl.ds(o, csz)])

            pl.when(sub < NS - 1)(lambda: writeout(sub * ZR, 4, SB))
            pl.when(sub == NS - 1)(lambda: writeout((NS - 1) * ZR, 5, 616))
            plsc.subcore_barrier()

    pl.when(core == 0)(lambda: side(hu8, sd_ub, sums_i, cnt_i))
    pl.when(core == 1)(lambda: side(hi8, sd_bu, sums_u, cnt_u))


@functools.partial(jax.jit, static_argnames=())
def _sc_aggregate(hu8, hi8, sd_ub, sd_bu):
    mesh = plsc.VectorSubcoreMesh(core_axis_name="c", subcore_axis_name="s")
    f32 = jnp.float32
    out_type = [
        jax.ShapeDtypeStruct((NP, N, 16), f32),
        jax.ShapeDtypeStruct((N, 16), f32),
        jax.ShapeDtypeStruct((NP, N, 16), f32),
        jax.ShapeDtypeStruct((N, 16), f32),
    ]
    scratch = [
        pltpu.VMEM_SHARED((ACC_R, 16), f32),
        pltpu.VMEM_SHARED((ACC_R, 16), f32),
        pltpu.VMEM((ZB, 16), f32),
        pltpu.VMEM((CH, 16), f32),
        pltpu.VMEM((2, CH), jnp.int32),
        pltpu.VMEM((2, CH), jnp.int32),
        pltpu.VMEM((CH, 16), f32),
        pltpu.VMEM((CH, 16), f32),
        pltpu.VMEM((SB, 16), f32),
        pltpu.SemaphoreType.DMA,
        pltpu.SemaphoreType.DMA,
        pltpu.SemaphoreType.DMA,
        pltpu.SemaphoreType.DMA,
    ]
    return pl.kernel(
        _sc_body, out_type=out_type, mesh=mesh, scratch_types=scratch,
        compiler_params=pltpu.CompilerParams(use_tc_tiling_on_sc=False),
        name="het_block_sc_aggregate",
    )(hu8, hi8, sd_ub, sd_bu)


def _tc_body(h_ref, sums_ref, cnt_ref, w_ref, b_ref, g_ref, be_ref, out_ref):
    rec = 1.0 / jnp.maximum(cnt_ref[:, 0:1], 1.0)
    y = jnp.dot(h_ref[...], w_ref[0:D, :], preferred_element_type=jnp.float32)
    for p in range(NP):
        y += jnp.dot(sums_ref[p] * rec, w_ref[D + 16 * p:D + 16 * (p + 1), :],
                     preferred_element_type=jnp.float32)
    y = y + b_ref[...]
    mu = jnp.mean(y, axis=-1, keepdims=True)
    yc = y - mu
    var = jnp.mean(yc * yc, axis=-1, keepdims=True)
    out_ref[...] = yc * lax.rsqrt(var + 1e-5) * g_ref[...] + be_ref[...]


def _tc_tail(h, sums, cnt, w, b, g, be):
    R = 2000
    grid = (N // R,)
    return pl.pallas_call(
        _tc_body,
        grid=grid,
        in_specs=[
            pl.BlockSpec((R, D), lambda i: (i, 0)),
            pl.BlockSpec((NP, R, 16), lambda i: (0, i, 0)),
            pl.BlockSpec((R, 16), lambda i: (i, 0)),
            pl.BlockSpec((2 * D, D), lambda i: (0, 0)),
            pl.BlockSpec((1, D), lambda i: (0, 0)),
            pl.BlockSpec((1, D), lambda i: (0, 0)),
            pl.BlockSpec((1, D), lambda i: (0, 0)),
        ],
        out_specs=pl.BlockSpec((R, D), lambda i: (i, 0)),
        out_shape=jax.ShapeDtypeStruct((N, D), jnp.float32),
    )(h, sums, cnt, w, b, g, be)


def kernel(h_user, h_item, edge_index_user_buys_item, edge_index_item_bought_by_user,
           W_ub, b_ub, W_bu, b_bu, g_user, be_user, g_item, be_item):
    i32 = jnp.int32
    pad = EPAD - E
    pad_s = (jnp.arange(pad, dtype=i32) % 64)
    pad_d = N + (jnp.arange(pad, dtype=i32) % 48)

    s_ub = jnp.concatenate([edge_index_user_buys_item[0], pad_s])
    d_ub = jnp.concatenate([edge_index_user_buys_item[1], pad_d])
    s_bu = jnp.concatenate([edge_index_item_bought_by_user[0], pad_s])
    d_bu = jnp.concatenate([edge_index_item_bought_by_user[1], pad_d])
    sd_ub = jnp.stack([s_ub.reshape(-1, CH), d_ub.reshape(-1, CH)], axis=1)
    sd_bu = jnp.stack([s_bu.reshape(-1, CH), d_bu.reshape(-1, CH)], axis=1)

    hu8 = h_user.reshape(N, NP, 16).transpose(1, 0, 2)
    hi8 = h_item.reshape(N, NP, 16).transpose(1, 0, 2)

    sums_i, cnt_i, sums_u, cnt_u = _sc_aggregate(hu8, hi8, sd_ub, sd_bu)

    out_item = _tc_tail(h_item, sums_i, cnt_i, W_ub, b_ub.reshape(1, D),
                        g_item.reshape(1, D), be_item.reshape(1, D))
    out_user = _tc_tail(h_user, sums_u, cnt_u, W_bu, b_bu.reshape(1, D),
                        g_user.reshape(1, D), be_user.reshape(1, D))
    return (out_user, out_item)


# speedup vs baseline: 2.1970x; 1.0959x over previous
import functools

import jax
import jax.numpy as jnp
from jax import lax
from jax.experimental import pallas as pl
from jax.experimental.pallas import tpu as pltpu
from jax.experimental.pallas import tpu_sc as plsc

N = 50000
D = 128
E = 500000
NS = 16
CH = 128
CPW = 248
EPW = CPW * CH
EPAD = NS * EPW
NP = D // 16
ACC_R = 50048
ZR = ACC_R // NS
ZB = 391
SB = 391


def _sc_body(hu8, hi8, sd_ub, sd_bu,
             sums_i, cnt_i, sums_u, cnt_u,
             acc, cacc, zbuf, ones,
             sd0, sd1, sd2, sd3, rows0, rows1, rows2, rows3, stage,
             semi0, semi1, semi2, semi3, semg0, semg1, semg2, semg3,
             sems0, sems1, sems2, sems3):
    core = lax.axis_index("c")
    sub = lax.axis_index("s")
    sd = (sd0, sd1, sd2, sd3)
    rows = (rows0, rows1, rows2, rows3)
    semi = (semi0, semi1, semi2, semi3)
    semg = (semg0, semg1, semg2, semg3)
    sems = (sems0, sems1, sems2, sems3)

    def fill(ref, n, val):
        def body(i, c):
            ref[i] = jnp.full((16,), val, jnp.float32)
            return c
        lax.fori_loop(0, n, body, 0)

    fill(zbuf, ZB, 0.0)
    fill(ones, CH, 1.0)

    def zero(dst):
        for k in range(ZR // ZB):
            pltpu.sync_copy(zbuf, dst.at[pl.ds(sub * ZR + k * ZB, ZB)])

    def side(tab8, sd_e, sums_o, cnt_o):
        zero(cacc)
        row0 = sub * CPW
        for p in range(NP):
            zero(acc)
            plsc.subcore_barrier()

            tab = tab8.at[p]

            def wait_i(b):
                pltpu.make_async_copy(sd_e.at[row0], sd[b], semi[b]).wait()

            def wait_g(b):
                pltpu.make_async_copy(tab.at[sd[b].at[0]], rows[b], semg[b]).wait()

            def wait_s(b):
                pltpu.make_async_copy(rows[b], acc.at[sd[b].at[1]], sems[b]).wait()

            def step(c, b, drain, prefetch, issue_gather):
                wait_g(b)
                if drain:
                    wait_s((b + 2) % 4)
                pltpu.async_copy(rows[b], acc.at[sd[b].at[1]], sems[b], add=True)
                if p == 0:
                    pltpu.sync_copy(ones, cacc.at[sd[b].at[1]], add=True)
                if prefetch:
                    pltpu.async_copy(sd_e.at[row0 + c + 2], sd[(b + 2) % 4],
                                     semi[(b + 2) % 4])
                if issue_gather:
                    bn = (b + 1) % 4
                    wait_i(bn)
                    pltpu.async_copy(tab.at[sd[bn].at[0]], rows[bn], semg[bn])

            pltpu.async_copy(sd_e.at[row0], sd0, semi0)
            pltpu.async_copy(sd_e.at[row0 + 1], sd1, semi1)
            wait_i(0)
            pltpu.async_copy(tab.at[sd0.at[0]], rows0, semg0)
            step(0, 0, False, True, True)
            step(1, 1, False, True, True)

            def body(j, c):
                for k in range(4):
                    cc = 4 * j + 2 + k
                    step(cc, (2 + k) % 4, True, True, True)
                return c

            lax.fori_loop(0, (CPW - 4) // 4, body, 0)
            step(CPW - 2, 2, True, False, True)
            step(CPW - 1, 3, True, False, False)
            wait_s(2)
            wait_s(3)
            plsc.subcore_barrier()

            def writeout(r0, nchunks, csz):
                for k in range(nchunks):
                    o = r0 + k * csz
                    pltpu.sync_copy(acc.at[pl.ds(o, csz)], stage.at[pl.ds(0, csz)])
                    pltpu.sync_copy(stage.at[pl.ds(0, csz)], sums_o.at[p, pl.ds(o, csz)])
                    if p == 0:
                        pltpu.sync_copy(cacc.at[pl.ds(o, csz)], stage.at[pl.ds(0, csz)])
                        pltpu.sync_copy(stage.at[pl.ds(0, csz)], cnt_o.at[pl.ds(o, csz)])

            pl.when(sub < NS - 1)(lambda: writeout(sub * ZR, 8, SB))
            pl.when(sub == NS - 1)(lambda: writeout((NS - 1) * ZR, 7, SB))
            pl.when(sub == NS - 1)(
                lambda: writeout((NS - 1) * ZR + 7 * SB, 1, 3080 - 7 * SB))
            plsc.subcore_barrier()

    pl.when(core == 0)(lambda: side(hu8, sd_ub, sums_i, cnt_i))
    pl.when(core == 1)(lambda: side(hi8, sd_bu, sums_u, cnt_u))


@functools.partial(jax.jit, static_argnames=())
def _sc_aggregate(hu8, hi8, sd_ub, sd_bu):
    mesh = plsc.VectorSubcoreMesh(core_axis_name="c", subcore_axis_name="s")
    f32 = jnp.float32
    out_type = [
        jax.ShapeDtypeStruct((NP, N, 16), f32),
        jax.ShapeDtypeStruct((N, 16), f32),
        jax.ShapeDtypeStruct((NP, N, 16), f32),
        jax.ShapeDtypeStruct((N, 16), f32),
    ]
    scratch = [
        pltpu.VMEM_SHARED((ACC_R, 16), f32),
        pltpu.VMEM_SHARED((ACC_R, 16), f32),
        pltpu.VMEM((ZB, 16), f32),
        pltpu.VMEM((CH, 16), f32),
        pltpu.VMEM((2, CH), jnp.int32),
        pltpu.VMEM((2, CH), jnp.int32),
        pltpu.VMEM((2, CH), jnp.int32),
        pltpu.VMEM((2, CH), jnp.int32),
        pltpu.VMEM((CH, 16), f32),
        pltpu.VMEM((CH, 16), f32),
        pltpu.VMEM((CH, 16), f32),
        pltpu.VMEM((CH, 16), f32),
        pltpu.VMEM((SB, 16), f32),
    ] + [pltpu.SemaphoreType.DMA] * 12
    return pl.kernel(
        _sc_body, out_type=out_type, mesh=mesh, scratch_types=scratch,
        compiler_params=pltpu.CompilerParams(use_tc_tiling_on_sc=False),
        name="het_block_sc_aggregate",
    )(hu8, hi8, sd_ub, sd_bu)


def _tc_body(h_ref, sums_ref, cnt_ref, w_ref, b_ref, g_ref, be_ref, out_ref):
    rec = 1.0 / jnp.maximum(cnt_ref[:, 0:1], 1.0)
    y = jnp.dot(h_ref[...], w_ref[0:D, :], preferred_element_type=jnp.float32)
    for p in range(NP):
        y += jnp.dot(sums_ref[p] * rec, w_ref[D + 16 * p:D + 16 * (p + 1), :],
                     preferred_element_type=jnp.float32)
    y = y + b_ref[...]
    mu = jnp.mean(y, axis=-1, keepdims=True)
    yc = y - mu
    var = jnp.mean(yc * yc, axis=-1, keepdims=True)
    out_ref[...] = yc * lax.rsqrt(var + 1e-5) * g_ref[...] + be_ref[...]


def _tc_tail(h, sums, cnt, w, b, g, be):
    R = 2000
    grid = (N // R,)
    return pl.pallas_call(
        _tc_body,
        grid=grid,
        in_specs=[
            pl.BlockSpec((R, D), lambda i: (i, 0)),
            pl.BlockSpec((NP, R, 16), lambda i: (0, i, 0)),
            pl.BlockSpec((R, 16), lambda i: (i, 0)),
            pl.BlockSpec((2 * D, D), lambda i: (0, 0)),
            pl.BlockSpec((1, D), lambda i: (0, 0)),
            pl.BlockSpec((1, D), lambda i: (0, 0)),
            pl.BlockSpec((1, D), lambda i: (0, 0)),
        ],
        out_specs=pl.BlockSpec((R, D), lambda i: (i, 0)),
        out_shape=jax.ShapeDtypeStruct((N, D), jnp.float32),
    )(h, sums, cnt, w, b, g, be)


def kernel(h_user, h_item, edge_index_user_buys_item, edge_index_item_bought_by_user,
           W_ub, b_ub, W_bu, b_bu, g_user, be_user, g_item, be_item):
    i32 = jnp.int32
    pad = EPAD - E
    pad_s = (jnp.arange(pad, dtype=i32) % 64)
    pad_d = N + (jnp.arange(pad, dtype=i32) % 48)

    s_ub = jnp.concatenate([edge_index_user_buys_item[0], pad_s])
    d_ub = jnp.concatenate([edge_index_user_buys_item[1], pad_d])
    s_bu = jnp.concatenate([edge_index_item_bought_by_user[0], pad_s])
    d_bu = jnp.concatenate([edge_index_item_bought_by_user[1], pad_d])
    sd_ub = jnp.stack([s_ub.reshape(-1, CH), d_ub.reshape(-1, CH)], axis=1)
    sd_bu = jnp.stack([s_bu.reshape(-1, CH), d_bu.reshape(-1, CH)], axis=1)

    hu8 = h_user.reshape(N, NP, 16).transpose(1, 0, 2)
    hi8 = h_item.reshape(N, NP, 16).transpose(1, 0, 2)

    sums_i, cnt_i, sums_u, cnt_u = _sc_aggregate(hu8, hi8, sd_ub, sd_bu)

    out_item = _tc_tail(h_item, sums_i, cnt_i, W_ub, b_ub.reshape(1, D),
                        g_item.reshape(1, D), be_item.reshape(1, D))
    out_user = _tc_tail(h_user, sums_u, cnt_u, W_bu, b_bu.reshape(1, D),
                        g_user.reshape(1, D), be_user.reshape(1, D))
    return (out_user, out_item)


# speedup vs baseline: 3.7077x; 1.6876x over previous
import functools

import jax
import jax.numpy as jnp
from jax import lax
from jax.experimental import pallas as pl
from jax.experimental.pallas import tpu as pltpu
from jax.experimental.pallas import tpu_sc as plsc

N = 50000
D = 128
E = 500000
NS = 16
CH = 128
CPW = 248
EPW = CPW * CH
EPAD = NS * EPW
CW = 32
NP = D // CW
ACC_R = 50048
ZR = ACC_R // NS
ZB = 136


def _sc_body(hu4, hi4, sd_ub, sd_bu,
             sums_i, cnt_i, sums_u, cnt_u,
             acc, zbuf, stage,
             sd0, sd1, sd2, sd3, rows0, rows1, rows2, rows3,
             semi0, semi1, semi2, semi3, semg0, semg1, semg2, semg3,
             sems0, sems1, sems2, sems3):
    core = lax.axis_index("c")
    sub = lax.axis_index("s")
    sd = (sd0, sd1, sd2, sd3)
    rows = (rows0, rows1, rows2, rows3)
    semi = (semi0, semi1, semi2, semi3)
    semg = (semg0, semg1, semg2, semg3)
    sems = (sems0, sems1, sems2, sems3)

    def fill(ref, n, val):
        def body(i, c):
            ref[i, 0:16] = jnp.full((16,), val, jnp.float32)
            ref[i, 16:32] = jnp.full((16,), val, jnp.float32)
            return c
        lax.fori_loop(0, n, body, 0)

    fill(zbuf, ZB, 0.0)

    def zero_acc():
        for k in range(ZR // ZB):
            pltpu.sync_copy(zbuf, acc.at[pl.ds(sub * ZR + k * ZB, ZB)])

    def writeout(dst, r0, nchunks, csz):
        for k in range(nchunks):
            o = r0 + k * csz
            pltpu.sync_copy(acc.at[pl.ds(o, csz)], stage.at[pl.ds(0, csz)])
            pltpu.sync_copy(stage.at[pl.ds(0, csz)], dst.at[pl.ds(o, csz)])

    def flush(dst):
        pl.when(sub < NS - 1)(lambda: writeout(dst, sub * ZR, 23, ZB))
        pl.when(sub == NS - 1)(lambda: writeout(dst, (NS - 1) * ZR, 22, ZB))
        pl.when(sub == NS - 1)(
            lambda: writeout(dst, (NS - 1) * ZR + 22 * ZB, 1, 3080 - 22 * ZB))

    def side(tab4, sd_e, sums_o, cnt_o):
        row0 = sub * CPW

        def run_pass(gather_p, dst_of_pass):
            def wait_i(b):
                pltpu.make_async_copy(sd_e.at[row0], sd[b], semi[b]).wait()

            def wait_g(b):
                pltpu.make_async_copy(tab4.at[0].at[sd[b].at[0]], rows[b],
                                      semg[b]).wait()

            def wait_s(b):
                pltpu.make_async_copy(rows[b], acc.at[sd[b].at[1]],
                                      sems[b]).wait()

            def step(c, b, drain, prefetch, issue_gather):
                if gather_p is not None:
                    wait_g(b)
                if drain:
                    wait_s((b + 2) % 4)
                src = rows[b] if gather_p is not None else rows0
                pltpu.async_copy(src, acc.at[sd[b].at[1]], sems[b], add=True)
                if prefetch:
                    pltpu.async_copy(sd_e.at[row0 + c + 2], sd[(b + 2) % 4],
                                     semi[(b + 2) % 4])
                if issue_gather and gather_p is not None:
                    bn = (b + 1) % 4
                    wait_i(bn)
                    pltpu.async_copy(gather_p.at[sd[bn].at[0]], rows[bn],
                                     semg[bn])
                if issue_gather and gather_p is None:
                    wait_i((b + 1) % 4)

            pltpu.async_copy(sd_e.at[row0], sd0, semi0)
            pltpu.async_copy(sd_e.at[row0 + 1], sd1, semi1)
            wait_i(0)
            if gather_p is not None:
                pltpu.async_copy(gather_p.at[sd0.at[0]], rows0, semg0)
            step(0, 0, False, True, True)
            step(1, 1, False, True, True)

            def body(j, c):
                for k in range(4):
                    step(4 * j + 2 + k, (2 + k) % 4, True, True, True)
                return c

            lax.fori_loop(0, (CPW - 4) // 4, body, 0)
            step(CPW - 2, 2, True, False, True)
            step(CPW - 1, 3, True, False, False)
            wait_s(2)
            wait_s(3)
            plsc.subcore_barrier()
            flush(dst_of_pass)
            plsc.subcore_barrier()

        for p in range(NP):
            zero_acc()
            plsc.subcore_barrier()
            run_pass(tab4.at[p], sums_o.at[p])

        zero_acc()
        fill(rows0, CH, 1.0)
        plsc.subcore_barrier()
        run_pass(None, cnt_o)

    pl.when(core == 0)(lambda: side(hu4, sd_ub, sums_i, cnt_i))
    pl.when(core == 1)(lambda: side(hi4, sd_bu, sums_u, cnt_u))


@functools.partial(jax.jit, static_argnames=())
def _sc_aggregate(hu4, hi4, sd_ub, sd_bu):
    mesh = plsc.VectorSubcoreMesh(core_axis_name="c", subcore_axis_name="s")
    f32 = jnp.float32
    out_type = [
        jax.ShapeDtypeStruct((NP, N, CW), f32),
        jax.ShapeDtypeStruct((N, CW), f32),
        jax.ShapeDtypeStruct((NP, N, CW), f32),
        jax.ShapeDtypeStruct((N, CW), f32),
    ]
    scratch = [
        pltpu.VMEM_SHARED((ACC_R, CW), f32),
        pltpu.VMEM((ZB, CW), f32),
        pltpu.VMEM((ZB, CW), f32),
        pltpu.VMEM((2, CH), jnp.int32),
        pltpu.VMEM((2, CH), jnp.int32),
        pltpu.VMEM((2, CH), jnp.int32),
        pltpu.VMEM((2, CH), jnp.int32),
        pltpu.VMEM((CH, CW), f32),
        pltpu.VMEM((CH, CW), f32),
        pltpu.VMEM((CH, CW), f32),
        pltpu.VMEM((CH, CW), f32),
    ] + [pltpu.SemaphoreType.DMA] * 12
    return pl.kernel(
        _sc_body, out_type=out_type, mesh=mesh, scratch_types=scratch,
        compiler_params=pltpu.CompilerParams(use_tc_tiling_on_sc=False),
        name="het_block_sc_aggregate",
    )(hu4, hi4, sd_ub, sd_bu)


def _tc_body(h_ref, sums_ref, cnt_ref, w_ref, b_ref, g_ref, be_ref, out_ref):
    rec = 1.0 / jnp.maximum(cnt_ref[:, 0:1], 1.0)
    y = jnp.dot(h_ref[...], w_ref[0:D, :], preferred_element_type=jnp.float32)
    for p in range(NP):
        y += jnp.dot(sums_ref[p] * rec, w_ref[D + CW * p:D + CW * (p + 1), :],
                     preferred_element_type=jnp.float32)
    y = y + b_ref[...]
    mu = jnp.mean(y, axis=-1, keepdims=True)
    yc = y - mu
    var = jnp.mean(yc * yc, axis=-1, keepdims=True)
    out_ref[...] = yc * lax.rsqrt(var + 1e-5) * g_ref[...] + be_ref[...]


def _tc_tail(h, sums, cnt, w, b, g, be):
    R = 2000
    grid = (N // R,)
    return pl.pallas_call(
        _tc_body,
        grid=grid,
        in_specs=[
            pl.BlockSpec((R, D), lambda i: (i, 0)),
            pl.BlockSpec((NP, R, CW), lambda i: (0, i, 0)),
            pl.BlockSpec((R, CW), lambda i: (i, 0)),
            pl.BlockSpec((2 * D, D), lambda i: (0, 0)),
            pl.BlockSpec((1, D), lambda i: (0, 0)),
            pl.BlockSpec((1, D), lambda i: (0, 0)),
            pl.BlockSpec((1, D), lambda i: (0, 0)),
        ],
        out_specs=pl.BlockSpec((R, D), lambda i: (i, 0)),
        out_shape=jax.ShapeDtypeStruct((N, D), jnp.float32),
    )(h, sums, cnt, w, b, g, be)


def kernel(h_user, h_item, edge_index_user_buys_item, edge_index_item_bought_by_user,
           W_ub, b_ub, W_bu, b_bu, g_user, be_user, g_item, be_item):
    i32 = jnp.int32
    pad = EPAD - E
    pad_s = (jnp.arange(pad, dtype=i32) % 64)
    pad_d = N + (jnp.arange(pad, dtype=i32) % 48)

    s_ub = jnp.concatenate([edge_index_user_buys_item[0], pad_s])
    d_ub = jnp.concatenate([edge_index_user_buys_item[1], pad_d])
    s_bu = jnp.concatenate([edge_index_item_bought_by_user[0], pad_s])
    d_bu = jnp.concatenate([edge_index_item_bought_by_user[1], pad_d])
    sd_ub = jnp.stack([s_ub.reshape(-1, CH), d_ub.reshape(-1, CH)], axis=1)
    sd_bu = jnp.stack([s_bu.reshape(-1, CH), d_bu.reshape(-1, CH)], axis=1)

    hu4 = h_user.reshape(N, NP, CW).transpose(1, 0, 2)
    hi4 = h_item.reshape(N, NP, CW).transpose(1, 0, 2)

    sums_i, cnt_i, sums_u, cnt_u = _sc_aggregate(hu4, hi4, sd_ub, sd_bu)

    out_item = _tc_tail(h_item, sums_i, cnt_i, W_ub, b_ub.reshape(1, D),
                        g_item.reshape(1, D), be_item.reshape(1, D))
    out_user = _tc_tail(h_user, sums_u, cnt_u, W_bu, b_bu.reshape(1, D),
                        g_user.reshape(1, D), be_user.reshape(1, D))
    return (out_user, out_item)


# speedup vs baseline: 4.6476x; 1.2535x over previous
import functools

import jax
import jax.numpy as jnp
from jax import lax
from jax.experimental import pallas as pl
from jax.experimental.pallas import tpu as pltpu
from jax.experimental.pallas import tpu_sc as plsc

N = 50000
D = 128
E = 500000
NS = 16
CH = 256
CPW = 124
EPW = CPW * CH
EPAD = NS * EPW
CW = 32
NP = D // CW
ACC_R = 50048
ZR = ACC_R // NS
ZB = 136


def _sc_body(hu4, hi4, sd_ub, sd_bu,
             sums_i, cnt_i, sums_u, cnt_u,
             acc, zbuf, stage,
             sd0, sd1, sd2, sd3, rows0, rows1,
             semi0, semi1, semi2, semi3, semg0, semg1,
             sems0, sems1, sems2, sems3):
    core = lax.axis_index("c")
    sub = lax.axis_index("s")
    sd = (sd0, sd1, sd2, sd3)
    rows = (rows0, rows1)
    semi = (semi0, semi1, semi2, semi3)
    semg = (semg0, semg1)
    sems = (sems0, sems1, sems2, sems3)

    def fill(ref, n, val):
        def body(i, c):
            ref[i, 0:16] = jnp.full((16,), val, jnp.float32)
            ref[i, 16:32] = jnp.full((16,), val, jnp.float32)
            return c
        lax.fori_loop(0, n, body, 0)

    fill(zbuf, ZB, 0.0)

    def zero_acc():
        for k in range(ZR // ZB):
            pltpu.sync_copy(zbuf, acc.at[pl.ds(sub * ZR + k * ZB, ZB)])

    def writeout(dst, r0, nchunks, csz):
        for k in range(nchunks):
            o = r0 + k * csz
            pltpu.sync_copy(acc.at[pl.ds(o, csz)], stage.at[pl.ds(0, csz)])
            pltpu.sync_copy(stage.at[pl.ds(0, csz)], dst.at[pl.ds(o, csz)])

    def flush(dst):
        pl.when(sub < NS - 1)(lambda: writeout(dst, sub * ZR, 23, ZB))
        pl.when(sub == NS - 1)(lambda: writeout(dst, (NS - 1) * ZR, 22, ZB))
        pl.when(sub == NS - 1)(
            lambda: writeout(dst, (NS - 1) * ZR + 22 * ZB, 1, 3080 - 22 * ZB))

    def side(tab4, sd_e, sums_o, cnt_o):
        row0 = sub * CPW

        def run_pass(gather_p, dst_of_pass):
            def wait_i(b4):
                pltpu.make_async_copy(sd_e.at[row0], sd[b4], semi[b4]).wait()

            def wait_g(b2):
                pltpu.make_async_copy(tab4.at[0].at[sd[0].at[0]], rows[b2],
                                      semg[b2]).wait()

            def wait_s(b4):
                pltpu.make_async_copy(rows[b4 % 2], acc.at[sd[b4].at[1]],
                                      sems[b4]).wait()

            def step(c, b2, b4, drain, prefetch, issue_gather):
                if gather_p is not None:
                    wait_g(b2)
                if drain:
                    wait_s((b4 + 3) % 4)
                src = rows[b2] if gather_p is not None else rows0
                pltpu.async_copy(src, acc.at[sd[b4].at[1]], sems[b4], add=True)
                if prefetch:
                    pltpu.async_copy(sd_e.at[row0 + c + 2], sd[(b4 + 2) % 4],
                                     semi[(b4 + 2) % 4])
                if issue_gather:
                    bn = (b4 + 1) % 4
                    wait_i(bn)
                    if gather_p is not None:
                        pltpu.async_copy(gather_p.at[sd[bn].at[0]],
                                         rows[1 - b2], semg[1 - b2])

            pltpu.async_copy(sd_e.at[row0], sd0, semi0)
            pltpu.async_copy(sd_e.at[row0 + 1], sd1, semi1)
            wait_i(0)
            if gather_p is not None:
                pltpu.async_copy(gather_p.at[sd0.at[0]], rows0, semg0)
            step(0, 0, 0, False, True, True)
            step(1, 1, 1, True, True, True)

            def body(j, c):
                for k in range(4):
                    step(4 * j + 2 + k, k % 2, (2 + k) % 4, True, True, True)
                return c

            lax.fori_loop(0, (CPW - 4) // 4, body, 0)
            step(CPW - 2, 0, 2, True, False, True)
            step(CPW - 1, 1, 3, True, False, False)
            wait_s(3)
            plsc.subcore_barrier()
            flush(dst_of_pass)
            plsc.subcore_barrier()

        for p in range(NP):
            zero_acc()
            plsc.subcore_barrier()
            run_pass(tab4.at[p], sums_o.at[p])

        zero_acc()
        fill(rows0, CH, 1.0)
        plsc.subcore_barrier()
        run_pass(None, cnt_o)

    pl.when(core == 0)(lambda: side(hu4, sd_ub, sums_i, cnt_i))
    pl.when(core == 1)(lambda: side(hi4, sd_bu, sums_u, cnt_u))


@functools.partial(jax.jit, static_argnames=())
def _sc_aggregate(hu4, hi4, sd_ub, sd_bu):
    mesh = plsc.VectorSubcoreMesh(core_axis_name="c", subcore_axis_name="s")
    f32 = jnp.float32
    out_type = [
        jax.ShapeDtypeStruct((NP, N, CW), f32),
        jax.ShapeDtypeStruct((N, CW), f32),
        jax.ShapeDtypeStruct((NP, N, CW), f32),
        jax.ShapeDtypeStruct((N, CW), f32),
    ]
    scratch = [
        pltpu.VMEM_SHARED((ACC_R, CW), f32),
        pltpu.VMEM((ZB, CW), f32),
        pltpu.VMEM((ZB, CW), f32),
        pltpu.VMEM((2, CH), jnp.int32),
        pltpu.VMEM((2, CH), jnp.int32),
        pltpu.VMEM((2, CH), jnp.int32),
        pltpu.VMEM((2, CH), jnp.int32),
        pltpu.VMEM((CH, CW), f32),
        pltpu.VMEM((CH, CW), f32),
    ] + [pltpu.SemaphoreType.DMA] * 10
    return pl.kernel(
        _sc_body, out_type=out_type, mesh=mesh, scratch_types=scratch,
        compiler_params=pltpu.CompilerParams(use_tc_tiling_on_sc=False),
        name="het_block_sc_aggregate",
    )(hu4, hi4, sd_ub, sd_bu)


def _tc_body(h_ref, sums_ref, cnt_ref, w_ref, b_ref, g_ref, be_ref, out_ref):
    rec = 1.0 / jnp.maximum(cnt_ref[:, 0:1], 1.0)
    y = jnp.dot(h_ref[...], w_ref[0:D, :], preferred_element_type=jnp.float32)
    for p in range(NP):
        y += jnp.dot(sums_ref[p] * rec, w_ref[D + CW * p:D + CW * (p + 1), :],
                     preferred_element_type=jnp.float32)
    y = y + b_ref[...]
    mu = jnp.mean(y, axis=-1, keepdims=True)
    yc = y - mu
    var = jnp.mean(yc * yc, axis=-1, keepdims=True)
    out_ref[...] = yc * lax.rsqrt(var + 1e-5) * g_ref[...] + be_ref[...]


def _tc_tail(h, sums, cnt, w, b, g, be):
    R = 2000
    grid = (N // R,)
    return pl.pallas_call(
        _tc_body,
        grid=grid,
        in_specs=[
            pl.BlockSpec((R, D), lambda i: (i, 0)),
            pl.BlockSpec((NP, R, CW), lambda i: (0, i, 0)),
            pl.BlockSpec((R, CW), lambda i: (i, 0)),
            pl.BlockSpec((2 * D, D), lambda i: (0, 0)),
            pl.BlockSpec((1, D), lambda i: (0, 0)),
            pl.BlockSpec((1, D), lambda i: (0, 0)),
            pl.BlockSpec((1, D), lambda i: (0, 0)),
        ],
        out_specs=pl.BlockSpec((R, D), lambda i: (i, 0)),
        out_shape=jax.ShapeDtypeStruct((N, D), jnp.float32),
    )(h, sums, cnt, w, b, g, be)


def kernel(h_user, h_item, edge_index_user_buys_item, edge_index_item_bought_by_user,
           W_ub, b_ub, W_bu, b_bu, g_user, be_user, g_item, be_item):
    i32 = jnp.int32
    pad = EPAD - E
    pad_s = (jnp.arange(pad, dtype=i32) % 64)
    pad_d = N + (jnp.arange(pad, dtype=i32) % 48)

    s_ub = jnp.concatenate([edge_index_user_buys_item[0], pad_s])
    d_ub = jnp.concatenate([edge_index_user_buys_item[1], pad_d])
    s_bu = jnp.concatenate([edge_index_item_bought_by_user[0], pad_s])
    d_bu = jnp.concatenate([edge_index_item_bought_by_user[1], pad_d])
    sd_ub = jnp.stack([s_ub.reshape(-1, CH), d_ub.reshape(-1, CH)], axis=1)
    sd_bu = jnp.stack([s_bu.reshape(-1, CH), d_bu.reshape(-1, CH)], axis=1)

    hu4 = h_user.reshape(N, NP, CW).transpose(1, 0, 2)
    hi4 = h_item.reshape(N, NP, CW).transpose(1, 0, 2)

    sums_i, cnt_i, sums_u, cnt_u = _sc_aggregate(hu4, hi4, sd_ub, sd_bu)

    out_item = _tc_tail(h_item, sums_i, cnt_i, W_ub, b_ub.reshape(1, D),
                        g_item.reshape(1, D), be_item.reshape(1, D))
    out_user = _tc_tail(h_user, sums_u, cnt_u, W_bu, b_bu.reshape(1, D),
                        g_user.reshape(1, D), be_user.reshape(1, D))
    return (out_user, out_item)


# speedup vs baseline: 4.7285x; 1.0174x over previous
import functools

import jax
import jax.numpy as jnp
from jax import lax
from jax.experimental import pallas as pl
from jax.experimental.pallas import tpu as pltpu
from jax.experimental.pallas import tpu_sc as plsc

N = 50000
D = 128
E = 500000
NS = 16
CH = 256
CPW = 124
EPW = CPW * CH
EPAD = NS * EPW
CW = 32
NP = D // CW
ACC_R = 50048
ZR = ACC_R // NS
ZB = 104


def _sc_body(hu4, hi4, sd_ub, sd_bu,
             sums_i, cnt_i, sums_u, cnt_u,
             acc, stage0, stage1,
             sd0, sd1, sd2, sd3, rows0, rows1,
             semi0, semi1, semi2, semi3, semg0, semg1,
             sems0, sems1, sems2, sems3, semf0, semf1):
    core = lax.axis_index("c")
    sub = lax.axis_index("s")
    sd = (sd0, sd1, sd2, sd3)
    rows = (rows0, rows1)
    stage = (stage0, stage1)
    semi = (semi0, semi1, semi2, semi3)
    semg = (semg0, semg1)
    sems = (sems0, sems1, sems2, sems3)
    semf = (semf0, semf1)

    def fill(ref, n, val):
        def body(i, c):
            ref[i, 0:16] = jnp.full((16,), val, jnp.float32)
            ref[i, 16:32] = jnp.full((16,), val, jnp.float32)
            return c
        lax.fori_loop(0, n, body, 0)

    def refill_zeros():
        fill(stage0, ZB, 0.0)
        fill(stage1, ZB, 0.0)

    refill_zeros()

    def zwait(dst, b):
        pltpu.make_async_copy(stage[b], dst.at[pl.ds(0, ZB)], semf[b]).wait()

    def zero_acc():
        def zstep(off, b, drain):
            if drain:
                zwait(acc, b)
            pltpu.async_copy(stage[b], acc.at[pl.ds(off, ZB)], semf[b])

        r0 = sub * ZR
        zstep(r0, 0, False)
        zstep(r0 + ZB, 1, False)

        def body(j, c):
            zstep(r0 + (2 * j) * ZB, 0, True)
            zstep(r0 + (2 * j + 1) * ZB, 1, True)
            return c

        lax.fori_loop(1, 15, body, 0)
        zstep(r0 + 3024, 0, True)
        zwait(acc, 1)
        zwait(acc, 0)

    def writeout(dst, r0, loop_hi, tail):
        def wstep(off, b, drain):
            if drain:
                zwait(dst, b)
            pltpu.sync_copy(acc.at[pl.ds(off, ZB)], stage[b])
            pltpu.async_copy(stage[b], dst.at[pl.ds(off, ZB)], semf[b])

        wstep(r0, 0, False)
        wstep(r0 + ZB, 1, False)

        def body(j, c):
            wstep(r0 + (2 * j) * ZB, 0, True)
            wstep(r0 + (2 * j + 1) * ZB, 1, True)
            return c

        lax.fori_loop(1, loop_hi, body, 0)
        for off, b in tail:
            wstep(r0 + off, b, True)
        zwait(dst, 0)
        zwait(dst, 1)

    def flush(dst):
        pl.when(sub < NS - 1)(lambda: writeout(
            dst, sub * ZR, 15, ((3024, 0),)))
        pl.when(sub == NS - 1)(lambda: writeout(
            dst, (NS - 1) * ZR, 14,
            ((28 * ZB, 0), (2976, 1))))
        refill_zeros()

    def side(tab4, sd_e, sums_o, cnt_o):
        row0 = sub * CPW

        def run_pass(gather_p, dst_of_pass):
            def wait_i(b4):
                pltpu.make_async_copy(sd_e.at[row0], sd[b4], semi[b4]).wait()

            def wait_g(b2):
                pltpu.make_async_copy(gather_p.at[sd[0].at[0]], rows[b2],
                                      semg[b2]).wait()

            def wait_s(b4):
                pltpu.make_async_copy(rows[b4 % 2], acc.at[sd[b4].at[1]],
                                      sems[b4]).wait()

            def step(c, b2, b4, drain, prefetch, issue_gather):
                if gather_p is not None:
                    wait_g(b2)
                if drain:
                    wait_s((b4 + 3) % 4)
                src = rows[b2] if gather_p is not None else rows0
                pltpu.async_copy(src, acc.at[sd[b4].at[1]], sems[b4], add=True)
                if prefetch:
                    pltpu.async_copy(sd_e.at[row0 + c + 2], sd[(b4 + 2) % 4],
                                     semi[(b4 + 2) % 4])
                if issue_gather:
                    bn = (b4 + 1) % 4
                    wait_i(bn)
                    if gather_p is not None:
                        pltpu.async_copy(gather_p.at[sd[bn].at[0]],
                                         rows[1 - b2], semg[1 - b2])

            pltpu.async_copy(sd_e.at[row0], sd0, semi0)
            pltpu.async_copy(sd_e.at[row0 + 1], sd1, semi1)
            wait_i(0)
            if gather_p is not None:
                pltpu.async_copy(gather_p.at[sd0.at[0]], rows0, semg0)
            step(0, 0, 0, False, True, True)
            step(1, 1, 1, True, True, True)

            def body(j, c):
                for k in range(4):
                    step(4 * j + 2 + k, k % 2, (2 + k) % 4, True, True, True)
                return c

            lax.fori_loop(0, (CPW - 4) // 4, body, 0)
            step(CPW - 2, 0, 2, True, False, True)
            step(CPW - 1, 1, 3, True, False, False)
            wait_s(3)
            plsc.subcore_barrier()
            flush(dst_of_pass)
            plsc.subcore_barrier()

        for p in range(NP):
            zero_acc()
            plsc.subcore_barrier()
            run_pass(tab4.at[p], sums_o.at[p])

        zero_acc()
        fill(rows0, CH, 1.0)
        plsc.subcore_barrier()
        run_pass(None, cnt_o)

    pl.when(core == 0)(lambda: side(hu4, sd_ub, sums_i, cnt_i))
    pl.when(core == 1)(lambda: side(hi4, sd_bu, sums_u, cnt_u))


@functools.partial(jax.jit, static_argnames=())
def _sc_aggregate(hu4, hi4, sd_ub, sd_bu):
    mesh = plsc.VectorSubcoreMesh(core_axis_name="c", subcore_axis_name="s")
    f32 = jnp.float32
    out_type = [
        jax.ShapeDtypeStruct((NP, N, CW), f32),
        jax.ShapeDtypeStruct((N, CW), f32),
        jax.ShapeDtypeStruct((NP, N, CW), f32),
        jax.ShapeDtypeStruct((N, CW), f32),
    ]
    scratch = [
        pltpu.VMEM_SHARED((ACC_R, CW), f32),
        pltpu.VMEM((ZB, CW), f32),
        pltpu.VMEM((ZB, CW), f32),
        pltpu.VMEM((2, CH), jnp.int32),
        pltpu.VMEM((2, CH), jnp.int32),
        pltpu.VMEM((2, CH), jnp.int32),
        pltpu.VMEM((2, CH), jnp.int32),
        pltpu.VMEM((CH, CW), f32),
        pltpu.VMEM((CH, CW), f32),
    ] + [pltpu.SemaphoreType.DMA] * 12
    return pl.kernel(
        _sc_body, out_type=out_type, mesh=mesh, scratch_types=scratch,
        compiler_params=pltpu.CompilerParams(use_tc_tiling_on_sc=False),
        name="het_block_sc_aggregate",
    )(hu4, hi4, sd_ub, sd_bu)


def _tc_body(h_ref, sums_ref, cnt_ref, w_ref, b_ref, g_ref, be_ref, out_ref):
    rec = 1.0 / jnp.maximum(cnt_ref[:, 0:1], 1.0)
    y = jnp.dot(h_ref[...], w_ref[0:D, :], preferred_element_type=jnp.float32)
    for p in range(NP):
        y += jnp.dot(sums_ref[p] * rec, w_ref[D + CW * p:D + CW * (p + 1), :],
                     preferred_element_type=jnp.float32)
    y = y + b_ref[...]
    mu = jnp.mean(y, axis=-1, keepdims=True)
    yc = y - mu
    var = jnp.mean(yc * yc, axis=-1, keepdims=True)
    out_ref[...] = yc * lax.rsqrt(var + 1e-5) * g_ref[...] + be_ref[...]


def _tc_tail(h, sums, cnt, w, b, g, be):
    R = 2000
    grid = (N // R,)
    return pl.pallas_call(
        _tc_body,
        grid=grid,
        in_specs=[
            pl.BlockSpec((R, D), lambda i: (i, 0)),
            pl.BlockSpec((NP, R, CW), lambda i: (0, i, 0)),
            pl.BlockSpec((R, CW), lambda i: (i, 0)),
            pl.BlockSpec((2 * D, D), lambda i: (0, 0)),
            pl.BlockSpec((1, D), lambda i: (0, 0)),
            pl.BlockSpec((1, D), lambda i: (0, 0)),
            pl.BlockSpec((1, D), lambda i: (0, 0)),
        ],
        out_specs=pl.BlockSpec((R, D), lambda i: (i, 0)),
        out_shape=jax.ShapeDtypeStruct((N, D), jnp.float32),
    )(h, sums, cnt, w, b, g, be)


def kernel(h_user, h_item, edge_index_user_buys_item, edge_index_item_bought_by_user,
           W_ub, b_ub, W_bu, b_bu, g_user, be_user, g_item, be_item):
    i32 = jnp.int32
    pad = EPAD - E
    pad_s = (jnp.arange(pad, dtype=i32) % 64)
    pad_d = N + (jnp.arange(pad, dtype=i32) % 48)

    s_ub = jnp.concatenate([edge_index_user_buys_item[0], pad_s])
    d_ub = jnp.concatenate([edge_index_user_buys_item[1], pad_d])
    s_bu = jnp.concatenate([edge_index_item_bought_by_user[0], pad_s])
    d_bu = jnp.concatenate([edge_index_item_bought_by_user[1], pad_d])
    sd_ub = jnp.stack([s_ub.reshape(-1, CH), d_ub.reshape(-1, CH)], axis=1)
    sd_bu = jnp.stack([s_bu.reshape(-1, CH), d_bu.reshape(-1, CH)], axis=1)

    hu4 = h_user.reshape(N, NP, CW).transpose(1, 0, 2)
    hi4 = h_item.reshape(N, NP, CW).transpose(1, 0, 2)

    sums_i, cnt_i, sums_u, cnt_u = _sc_aggregate(hu4, hi4, sd_ub, sd_bu)

    out_item = _tc_tail(h_item, sums_i, cnt_i, W_ub, b_ub.reshape(1, D),
                        g_item.reshape(1, D), be_item.reshape(1, D))
    out_user = _tc_tail(h_user, sums_u, cnt_u, W_bu, b_bu.reshape(1, D),
                        g_user.reshape(1, D), be_user.reshape(1, D))
    return (out_user, out_item)
